# initial kernel scaffold (unmeasured)
import jax
import jax.numpy as jnp
import numpy as np
from jax import lax
from jax.experimental import pallas as pl
from jax.experimental.pallas import tpu as pltpu

N_DEV = 8
SQ = 2048
D = 1024
HQ = 8
DH = 128
BLK = 512
CHUNK = SQ // N_DEV
SCALE = 0.08838834764831843
N_HOP = N_DEV - 1


def _rope_tables():
    inv = 1.0 / (10000.0 ** (np.arange(0, DH, 2) / DH))
    pos = np.arange(SQ)[:, None] * inv[None, :]
    cos = np.repeat(np.cos(pos), 2, axis=-1).astype(np.float32)
    sin = np.repeat(np.sin(pos), 2, axis=-1).astype(np.float32)
    P = np.zeros((DH, DH), np.float32)
    for k in range(DH // 2):
        P[2 * k + 1, 2 * k] = -1.0
        P[2 * k, 2 * k + 1] = 1.0
    return cos, sin, P


_COS, _SIN, _P = _rope_tables()


def kernel(x, Wq, Wk, Wv, Wo):
    xb = x.reshape(SQ, D).astype(jnp.bfloat16)
    wq = Wq.astype(jnp.bfloat16)
    wk = Wk.astype(jnp.bfloat16)
    wv = Wv.astype(jnp.bfloat16)
    wo = Wo.astype(jnp.bfloat16)
    cos = jnp.asarray(_COS)
    sin = jnp.asarray(_SIN)
    pmat = jnp.asarray(_P, jnp.bfloat16)

    def body(x_ref, wq_ref, wk_ref, wv_ref, wo_ref, cos_ref, sin_ref, p_ref,
             out_ref, q_ref, k_ref, v_ref, ctx_ref, comm_ref,
             send_sems, recv_sems):
        me = lax.axis_index("i")
        right = jnp.remainder(me + 1, N_DEV)

        xv = x_ref[...]
        q_ref[...] = jnp.dot(
            xv, wq_ref[...], preferred_element_type=jnp.float32
        ).astype(jnp.bfloat16)
        k_ref[...] = jnp.dot(
            xv, wk_ref[...], preferred_element_type=jnp.float32
        ).astype(jnp.bfloat16)
        v_ref[...] = jnp.dot(
            xv, wv_ref[...], preferred_element_type=jnp.float32
        ).astype(jnp.bfloat16)

        pm = p_ref[...]
        cos_f = cos_ref[...]
        sin_f = sin_ref[...]
        for h in range(HQ):
            c0 = h * DH
            kh = k_ref[:, c0:c0 + DH]
            krot = (
                kh.astype(jnp.float32) * cos_f
                + jnp.dot(kh, pm, preferred_element_type=jnp.float32) * sin_f
            ).astype(jnp.bfloat16)
            vh = v_ref[:, c0:c0 + DH]
            for rb in range(SQ // BLK):
                r0 = rb * BLK
                qh = q_ref[r0:r0 + BLK, c0:c0 + DH]
                qrot = (
                    (
                        qh.astype(jnp.float32) * cos_f[r0:r0 + BLK, :]
                        + jnp.dot(qh, pm, preferred_element_type=jnp.float32)
                        * sin_f[r0:r0 + BLK, :]
                    )
                    * SCALE
                ).astype(jnp.bfloat16)
                s = lax.dot_general(
                    qrot, krot, (((1,), (1,)), ((), ())),
                    preferred_element_type=jnp.float32,
                )
                mx = jnp.max(s, axis=1, keepdims=True)
                e = jnp.exp(s - mx)
                w = (e / jnp.sum(e, axis=1, keepdims=True)).astype(jnp.bfloat16)
                ctx_ref[r0:r0 + BLK, c0:c0 + DH] = jnp.dot(
                    w, vh, preferred_element_type=jnp.float32
                ).astype(jnp.bfloat16)

        out_ref[...] = jnp.dot(
            ctx_ref[...], wo_ref[...], preferred_element_type=jnp.float32
        )

        for hop in range(N_HOP):
            c_send = jnp.remainder(me - hop, N_DEV)
            c_recv = jnp.remainder(me - hop - 1, N_DEV)
            rdma = pltpu.make_async_remote_copy(
                src_ref=out_ref.at[pl.ds(c_send * CHUNK, CHUNK), :],
                dst_ref=comm_ref.at[hop],
                send_sem=send_sems.at[hop],
                recv_sem=recv_sems.at[hop],
                device_id=(right,),
                device_id_type=pl.DeviceIdType.MESH,
            )
            rdma.start()
            rdma.wait()
            sl = pl.ds(c_recv * CHUNK, CHUNK)
            out_ref[sl, :] = out_ref[sl, :] + comm_ref[hop]

        for g in range(N_HOP):
            idx = jnp.remainder(me + 1 - g, N_DEV)
            sl = pl.ds(idx * CHUNK, CHUNK)
            rdma = pltpu.make_async_remote_copy(
                src_ref=out_ref.at[sl, :],
                dst_ref=out_ref.at[sl, :],
                send_sem=send_sems.at[N_HOP + g],
                recv_sem=recv_sems.at[N_HOP + g],
                device_id=(right,),
                device_id_type=pl.DeviceIdType.MESH,
            )
            rdma.start()
            rdma.wait()

    out = pl.pallas_call(
        body,
        out_shape=jax.ShapeDtypeStruct((SQ, D), jnp.float32),
        in_specs=[pl.BlockSpec(memory_space=pltpu.VMEM)] * 8,
        out_specs=pl.BlockSpec(memory_space=pltpu.VMEM),
        scratch_shapes=[
            pltpu.VMEM((SQ, D), jnp.bfloat16),
            pltpu.VMEM((SQ, D), jnp.bfloat16),
            pltpu.VMEM((SQ, D), jnp.bfloat16),
            pltpu.VMEM((SQ, D), jnp.bfloat16),
            pltpu.VMEM((N_HOP, CHUNK, D), jnp.float32),
            pltpu.SemaphoreType.DMA((2 * N_HOP,)),
            pltpu.SemaphoreType.DMA((2 * N_HOP,)),
        ],
        compiler_params=pltpu.CompilerParams(collective_id=0),
    )(xb, wq, wk, wv, wo, cos, sin, pmat)
    return out.reshape(1, SQ, D)


# baseline (device time: 305343 ns/iter reference)
import jax
import jax.numpy as jnp
import numpy as np
from jax import lax
from jax.experimental import pallas as pl
from jax.experimental.pallas import tpu as pltpu

N_DEV = 8
SQ = 2048
D = 1024
HQ = 8
DH = 128
BLK = 512
CHUNK = SQ // N_DEV
SCALE = 0.08838834764831843
N_HOP = N_DEV - 1


def _rope_tables():
    inv = 1.0 / (10000.0 ** (np.arange(0, DH, 2) / DH))
    pos = np.arange(SQ)[:, None] * inv[None, :]
    cos = np.repeat(np.cos(pos), 2, axis=-1).astype(np.float32)
    sin = np.repeat(np.sin(pos), 2, axis=-1).astype(np.float32)
    P = np.zeros((DH, DH), np.float32)
    for k in range(DH // 2):
        P[2 * k + 1, 2 * k] = -1.0
        P[2 * k, 2 * k + 1] = 1.0
    return cos, sin, P


_COS, _SIN, _P = _rope_tables()


def kernel(x, Wq, Wk, Wv, Wo):
    xb = x.reshape(SQ, D).astype(jnp.bfloat16)
    wq = Wq.astype(jnp.bfloat16)
    wk = Wk.astype(jnp.bfloat16)
    wv = Wv.astype(jnp.bfloat16)
    wo = Wo.astype(jnp.bfloat16)
    cos = jnp.asarray(_COS)
    sin = jnp.asarray(_SIN)
    pmat = jnp.asarray(_P, jnp.bfloat16)

    def body(x_ref, wq_ref, wk_ref, wv_ref, wo_ref, cos_ref, sin_ref, p_ref,
             out_ref, q_ref, k_ref, v_ref, ctx_ref, comm_ref,
             send_sems, recv_sems):
        me = lax.axis_index("i")
        right = jnp.remainder(me + 1, N_DEV)

        xv = x_ref[...]
        q_ref[...] = jnp.dot(
            xv, wq_ref[...], preferred_element_type=jnp.float32
        ).astype(jnp.bfloat16)
        k_ref[...] = jnp.dot(
            xv, wk_ref[...], preferred_element_type=jnp.float32
        ).astype(jnp.bfloat16)
        v_ref[...] = jnp.dot(
            xv, wv_ref[...], preferred_element_type=jnp.float32
        ).astype(jnp.bfloat16)

        pm = p_ref[...]
        cos_f = cos_ref[...]
        sin_f = sin_ref[...]
        for h in range(HQ):
            c0 = h * DH
            kh = k_ref[:, c0:c0 + DH]
            krot = (
                kh.astype(jnp.float32) * cos_f
                + jnp.dot(kh, pm, preferred_element_type=jnp.float32) * sin_f
            ).astype(jnp.bfloat16)
            vh = v_ref[:, c0:c0 + DH]
            for rb in range(SQ // BLK):
                r0 = rb * BLK
                qh = q_ref[r0:r0 + BLK, c0:c0 + DH]
                qrot = (
                    (
                        qh.astype(jnp.float32) * cos_f[r0:r0 + BLK, :]
                        + jnp.dot(qh, pm, preferred_element_type=jnp.float32)
                        * sin_f[r0:r0 + BLK, :]
                    )
                    * SCALE
                ).astype(jnp.bfloat16)
                s = lax.dot_general(
                    qrot, krot, (((1,), (1,)), ((), ())),
                    preferred_element_type=jnp.float32,
                )
                mx = jnp.max(s, axis=1, keepdims=True)
                e = jnp.exp(s - mx)
                w = (e / jnp.sum(e, axis=1, keepdims=True)).astype(jnp.bfloat16)
                ctx_ref[r0:r0 + BLK, c0:c0 + DH] = jnp.dot(
                    w, vh, preferred_element_type=jnp.float32
                ).astype(jnp.bfloat16)

        out_ref[...] = jnp.dot(
            ctx_ref[...], wo_ref[...], preferred_element_type=jnp.float32
        )

        for hop in range(N_HOP):
            c_send = jnp.remainder(me - hop, N_DEV)
            c_recv = jnp.remainder(me - hop - 1, N_DEV)
            rdma = pltpu.make_async_remote_copy(
                src_ref=out_ref.at[pl.ds(c_send * CHUNK, CHUNK), :],
                dst_ref=comm_ref.at[hop],
                send_sem=send_sems.at[hop],
                recv_sem=recv_sems.at[hop],
                device_id=(right,),
                device_id_type=pl.DeviceIdType.MESH,
            )
            rdma.start()
            rdma.wait()
            sl = pl.ds(c_recv * CHUNK, CHUNK)
            out_ref[sl, :] = out_ref[sl, :] + comm_ref[hop]

        for g in range(N_HOP):
            idx = jnp.remainder(me + 1 - g, N_DEV)
            sl = pl.ds(idx * CHUNK, CHUNK)
            rdma = pltpu.make_async_remote_copy(
                src_ref=out_ref.at[sl, :],
                dst_ref=out_ref.at[sl, :],
                send_sem=send_sems.at[N_HOP + g],
                recv_sem=recv_sems.at[N_HOP + g],
                device_id=(right,),
                device_id_type=pl.DeviceIdType.MESH,
            )
            rdma.start()
            rdma.wait()

    out = pl.pallas_call(
        body,
        out_shape=jax.ShapeDtypeStruct((SQ, D), jnp.float32),
        in_specs=[pl.BlockSpec(memory_space=pltpu.VMEM)] * 8,
        out_specs=pl.BlockSpec(memory_space=pltpu.VMEM),
        scratch_shapes=[
            pltpu.VMEM((SQ, D), jnp.bfloat16),
            pltpu.VMEM((SQ, D), jnp.bfloat16),
            pltpu.VMEM((SQ, D), jnp.bfloat16),
            pltpu.VMEM((SQ, D), jnp.bfloat16),
            pltpu.VMEM((N_HOP, CHUNK, D), jnp.float32),
            pltpu.SemaphoreType.DMA((2 * N_HOP,)),
            pltpu.SemaphoreType.DMA((2 * N_HOP,)),
        ],
        compiler_params=pltpu.CompilerParams(
            vmem_limit_bytes=100 * 1024 * 1024,
        ),
    )(xb, wq, wk, wv, wo, cos, sin, pmat)
    return out.reshape(1, SQ, D)


# device time: 202649 ns/iter; 1.5068x vs baseline; 1.5068x over previous
import jax
import jax.numpy as jnp
import numpy as np
from jax import lax
from jax.experimental import pallas as pl
from jax.experimental.pallas import tpu as pltpu

N_DEV = 8
SQ = 2048
D = 1024
HQ = 8
DH = 128
BLK = 512
CHUNK = SQ // N_DEV
SCALE = 0.08838834764831843
N_HOP = N_DEV - 1


def _rope_tables():
    inv = 1.0 / (10000.0 ** (np.arange(0, DH, 2) / DH))
    pos = np.arange(SQ)[:, None] * inv[None, :]
    cos = np.repeat(np.cos(pos), 2, axis=-1).astype(np.float32)
    sin = np.repeat(np.sin(pos), 2, axis=-1).astype(np.float32)
    P = np.zeros((DH, DH), np.float32)
    for k in range(DH // 2):
        P[2 * k + 1, 2 * k] = -1.0
        P[2 * k, 2 * k + 1] = 1.0
    return cos, sin, P


_COS, _SIN, _P = _rope_tables()


def kernel(x, Wq, Wk, Wv, Wo):
    xb = x.reshape(SQ, D).astype(jnp.bfloat16)
    wq = Wq.astype(jnp.bfloat16)
    wk = Wk.astype(jnp.bfloat16)
    wv = Wv.astype(jnp.bfloat16)
    wo = Wo.astype(jnp.bfloat16)
    cos = jnp.asarray(_COS)
    sin = jnp.asarray(_SIN)
    pmat = jnp.asarray(_P, jnp.bfloat16)

    def body(x_ref, wq_ref, wk_ref, wv_ref, wo_ref, cos_ref, sin_ref, p_ref,
             out_ref, q_ref, k_ref, v_ref, ctx_ref, sbuf_ref, comm_ref,
             res_ref, send_sems, recv_sems):
        me = lax.axis_index("i")
        right = jnp.remainder(me + 1, N_DEV)

        xv = x_ref[...]
        q_ref[...] = jnp.dot(
            xv, wq_ref[...], preferred_element_type=jnp.float32
        ).astype(jnp.bfloat16)
        k_ref[...] = jnp.dot(
            xv, wk_ref[...], preferred_element_type=jnp.float32
        ).astype(jnp.bfloat16)
        v_ref[...] = jnp.dot(
            xv, wv_ref[...], preferred_element_type=jnp.float32
        ).astype(jnp.bfloat16)

        pm = p_ref[...]
        cos_f = cos_ref[...]
        sin_f = sin_ref[...]
        for h in range(HQ):
            c0 = h * DH
            kh = k_ref[:, c0:c0 + DH]
            krot = (
                kh.astype(jnp.float32) * cos_f
                + jnp.dot(kh, pm, preferred_element_type=jnp.float32) * sin_f
            ).astype(jnp.bfloat16)
            vh = v_ref[:, c0:c0 + DH]
            for rb in range(SQ // BLK):
                r0 = rb * BLK
                qh = q_ref[r0:r0 + BLK, c0:c0 + DH]
                qrot = (
                    (
                        qh.astype(jnp.float32) * cos_f[r0:r0 + BLK, :]
                        + jnp.dot(qh, pm, preferred_element_type=jnp.float32)
                        * sin_f[r0:r0 + BLK, :]
                    )
                    * SCALE
                ).astype(jnp.bfloat16)
                s = lax.dot_general(
                    qrot, krot, (((1,), (1,)), ((), ())),
                    preferred_element_type=jnp.float32,
                )
                e = jnp.exp(s)
                denom = jnp.sum(e, axis=1, keepdims=True)
                pv = jnp.dot(
                    e.astype(jnp.bfloat16), vh,
                    preferred_element_type=jnp.float32,
                )
                ctx_ref[r0:r0 + BLK, c0:c0 + DH] = (pv / denom).astype(
                    jnp.bfloat16
                )

        out_ref[...] = jnp.dot(
            ctx_ref[...], wo_ref[...], preferred_element_type=jnp.float32
        )

        for hop in range(N_HOP):
            c_send = jnp.remainder(me - hop, N_DEV)
            c_recv = jnp.remainder(me - hop - 1, N_DEV)
            ssl = pl.ds(c_send * CHUNK, CHUNK)
            sbuf_ref[hop] = out_ref[ssl, :].astype(jnp.bfloat16)
            rdma = pltpu.make_async_remote_copy(
                src_ref=sbuf_ref.at[hop],
                dst_ref=comm_ref.at[hop],
                send_sem=send_sems.at[hop],
                recv_sem=recv_sems.at[hop],
                device_id=(right,),
                device_id_type=pl.DeviceIdType.MESH,
            )
            rdma.start()
            rdma.wait()
            sl = pl.ds(c_recv * CHUNK, CHUNK)
            out_ref[sl, :] = out_ref[sl, :] + comm_ref[hop].astype(
                jnp.float32
            )

        own = jnp.remainder(me + 1, N_DEV)
        osl = pl.ds(own * CHUNK, CHUNK)
        res_ref[osl, :] = out_ref[osl, :].astype(jnp.bfloat16)
        for g in range(N_HOP):
            idx = jnp.remainder(me + 1 - g, N_DEV)
            sl = pl.ds(idx * CHUNK, CHUNK)
            rdma = pltpu.make_async_remote_copy(
                src_ref=res_ref.at[sl, :],
                dst_ref=res_ref.at[sl, :],
                send_sem=send_sems.at[N_HOP + g],
                recv_sem=recv_sems.at[N_HOP + g],
                device_id=(right,),
                device_id_type=pl.DeviceIdType.MESH,
            )
            rdma.start()
            rdma.wait()
            rsl = pl.ds(jnp.remainder(me - g, N_DEV) * CHUNK, CHUNK)
            out_ref[rsl, :] = res_ref[rsl, :].astype(jnp.float32)

    out = pl.pallas_call(
        body,
        out_shape=jax.ShapeDtypeStruct((SQ, D), jnp.float32),
        in_specs=[pl.BlockSpec(memory_space=pltpu.VMEM)] * 8,
        out_specs=pl.BlockSpec(memory_space=pltpu.VMEM),
        scratch_shapes=[
            pltpu.VMEM((SQ, D), jnp.bfloat16),
            pltpu.VMEM((SQ, D), jnp.bfloat16),
            pltpu.VMEM((SQ, D), jnp.bfloat16),
            pltpu.VMEM((SQ, D), jnp.bfloat16),
            pltpu.VMEM((N_HOP, CHUNK, D), jnp.bfloat16),
            pltpu.VMEM((N_HOP, CHUNK, D), jnp.bfloat16),
            pltpu.VMEM((SQ, D), jnp.bfloat16),
            pltpu.SemaphoreType.DMA((2 * N_HOP,)),
            pltpu.SemaphoreType.DMA((2 * N_HOP,)),
        ],
        compiler_params=pltpu.CompilerParams(
            vmem_limit_bytes=100 * 1024 * 1024,
        ),
    )(xb, wq, wk, wv, wo, cos, sin, pmat)
    return out.reshape(1, SQ, D)


# device time: 168611 ns/iter; 1.8109x vs baseline; 1.2019x over previous
import jax
import jax.numpy as jnp
import numpy as np
from jax import lax
from jax.experimental import pallas as pl
from jax.experimental.pallas import tpu as pltpu

N_DEV = 8
SQ = 2048
D = 1024
HQ = 8
DH = 128
BLK = 512
CHUNK = SQ // N_DEV
HALF = D // 2
SCALE = 0.08838834764831843
N_HOP = N_DEV - 1


def _rope_tables():
    inv = 1.0 / (10000.0 ** (np.arange(0, DH, 2) / DH))
    pos = np.arange(SQ)[:, None] * inv[None, :]
    cos = np.repeat(np.cos(pos), 2, axis=-1).astype(np.float32)
    sin = np.repeat(np.sin(pos), 2, axis=-1).astype(np.float32)
    P = np.zeros((DH, DH), np.float32)
    for k in range(DH // 2):
        P[2 * k + 1, 2 * k] = -1.0
        P[2 * k, 2 * k + 1] = 1.0
    return cos, sin, P


_COS, _SIN, _P = _rope_tables()


def kernel(x, Wq, Wk, Wv, Wo):
    xb = x.reshape(SQ, D).astype(jnp.bfloat16)
    wq = Wq.astype(jnp.bfloat16)
    wk = Wk.astype(jnp.bfloat16)
    wv = Wv.astype(jnp.bfloat16)
    wo = Wo.astype(jnp.bfloat16)
    cos = jnp.asarray(_COS)
    sin = jnp.asarray(_SIN)
    pmat = jnp.asarray(_P, jnp.bfloat16)

    def body(x_ref, wq_ref, wk_ref, wv_ref, wo_ref, cos_ref, sin_ref, p_ref,
             out_ref, q_ref, k_ref, v_ref, ctx_ref, sbuf_p, comm_p,
             sbuf_m, comm_m, agstage_p, agcomm_p, agstage_m, agcomm_m,
             send_sems_p, recv_sems_p, send_sems_m, recv_sems_m):
        me = lax.axis_index("i")
        right = jnp.remainder(me + 1, N_DEV)

        xv = x_ref[...]
        q_ref[...] = jnp.dot(
            xv, wq_ref[...], preferred_element_type=jnp.float32
        ).astype(jnp.bfloat16)
        k_ref[...] = jnp.dot(
            xv, wk_ref[...], preferred_element_type=jnp.float32
        ).astype(jnp.bfloat16)
        v_ref[...] = jnp.dot(
            xv, wv_ref[...], preferred_element_type=jnp.float32
        ).astype(jnp.bfloat16)

        pm = p_ref[...]
        cos_f = cos_ref[...]
        sin_f = sin_ref[...]
        for h in range(HQ):
            c0 = h * DH
            kh = k_ref[:, c0:c0 + DH]
            krot = (
                kh.astype(jnp.float32) * cos_f
                + jnp.dot(kh, pm, preferred_element_type=jnp.float32) * sin_f
            ).astype(jnp.bfloat16)
            vh = v_ref[:, c0:c0 + DH]
            for rb in range(SQ // BLK):
                r0 = rb * BLK
                qh = q_ref[r0:r0 + BLK, c0:c0 + DH]
                qrot = (
                    (
                        qh.astype(jnp.float32) * cos_f[r0:r0 + BLK, :]
                        + jnp.dot(qh, pm, preferred_element_type=jnp.float32)
                        * sin_f[r0:r0 + BLK, :]
                    )
                    * SCALE
                ).astype(jnp.bfloat16)
                s = lax.dot_general(
                    qrot, krot, (((1,), (1,)), ((), ())),
                    preferred_element_type=jnp.float32,
                )
                e = jnp.exp(s)
                denom = jnp.sum(e, axis=1, keepdims=True)
                pv = jnp.dot(
                    e.astype(jnp.bfloat16), vh,
                    preferred_element_type=jnp.float32,
                )
                ctx_ref[r0:r0 + BLK, c0:c0 + DH] = (pv / denom).astype(
                    jnp.bfloat16
                )

        out_ref[...] = jnp.dot(
            ctx_ref[...], wo_ref[...], preferred_element_type=jnp.float32
        )

        left = jnp.remainder(me - 1, N_DEV)

        for hop in range(N_HOP):
            p_send = pl.ds(jnp.remainder(me - hop, N_DEV) * CHUNK, CHUNK)
            m_send = pl.ds(jnp.remainder(me + hop, N_DEV) * CHUNK, CHUNK)
            sbuf_p[hop] = out_ref[p_send, 0:HALF].astype(jnp.bfloat16)
            sbuf_m[hop] = out_ref[m_send, HALF:D].astype(jnp.bfloat16)
            rdma_p = pltpu.make_async_remote_copy(
                src_ref=sbuf_p.at[hop],
                dst_ref=comm_p.at[hop],
                send_sem=send_sems_p.at[hop],
                recv_sem=recv_sems_p.at[hop],
                device_id=(right,),
                device_id_type=pl.DeviceIdType.MESH,
            )
            rdma_m = pltpu.make_async_remote_copy(
                src_ref=sbuf_m.at[hop],
                dst_ref=comm_m.at[hop],
                send_sem=send_sems_m.at[hop],
                recv_sem=recv_sems_m.at[hop],
                device_id=(left,),
                device_id_type=pl.DeviceIdType.MESH,
            )
            rdma_p.start()
            rdma_m.start()
            rdma_p.wait()
            rdma_m.wait()
            p_recv = pl.ds(
                jnp.remainder(me - hop - 1, N_DEV) * CHUNK, CHUNK
            )
            m_recv = pl.ds(
                jnp.remainder(me + hop + 1, N_DEV) * CHUNK, CHUNK
            )
            out_ref[p_recv, 0:HALF] = out_ref[p_recv, 0:HALF] + comm_p[
                hop
            ].astype(jnp.float32)
            out_ref[m_recv, HALF:D] = out_ref[m_recv, HALF:D] + comm_m[
                hop
            ].astype(jnp.float32)

        own_p = pl.ds(jnp.remainder(me + 1, N_DEV) * CHUNK, CHUNK)
        own_m = pl.ds(jnp.remainder(me - 1, N_DEV) * CHUNK, CHUNK)
        agstage_p[...] = out_ref[own_p, 0:HALF].astype(jnp.bfloat16)
        agstage_m[...] = out_ref[own_m, HALF:D].astype(jnp.bfloat16)
        for g in range(N_HOP):
            src_p = agstage_p if g == 0 else agcomm_p.at[g - 1]
            src_m = agstage_m if g == 0 else agcomm_m.at[g - 1]
            rdma_p = pltpu.make_async_remote_copy(
                src_ref=src_p,
                dst_ref=agcomm_p.at[g],
                send_sem=send_sems_p.at[N_HOP + g],
                recv_sem=recv_sems_p.at[N_HOP + g],
                device_id=(right,),
                device_id_type=pl.DeviceIdType.MESH,
            )
            rdma_m = pltpu.make_async_remote_copy(
                src_ref=src_m,
                dst_ref=agcomm_m.at[g],
                send_sem=send_sems_m.at[N_HOP + g],
                recv_sem=recv_sems_m.at[N_HOP + g],
                device_id=(left,),
                device_id_type=pl.DeviceIdType.MESH,
            )
            rdma_p.start()
            rdma_m.start()
            rdma_p.wait()
            rdma_m.wait()
            r_p = pl.ds(jnp.remainder(me - g, N_DEV) * CHUNK, CHUNK)
            r_m = pl.ds(jnp.remainder(me + g, N_DEV) * CHUNK, CHUNK)
            out_ref[r_p, 0:HALF] = agcomm_p[g].astype(jnp.float32)
            out_ref[r_m, HALF:D] = agcomm_m[g].astype(jnp.float32)

    out = pl.pallas_call(
        body,
        out_shape=jax.ShapeDtypeStruct((SQ, D), jnp.float32),
        in_specs=[pl.BlockSpec(memory_space=pltpu.VMEM)] * 8,
        out_specs=pl.BlockSpec(memory_space=pltpu.VMEM),
        scratch_shapes=[
            pltpu.VMEM((SQ, D), jnp.bfloat16),
            pltpu.VMEM((SQ, D), jnp.bfloat16),
            pltpu.VMEM((SQ, D), jnp.bfloat16),
            pltpu.VMEM((SQ, D), jnp.bfloat16),
            pltpu.VMEM((N_HOP, CHUNK, HALF), jnp.bfloat16),
            pltpu.VMEM((N_HOP, CHUNK, HALF), jnp.bfloat16),
            pltpu.VMEM((N_HOP, CHUNK, HALF), jnp.bfloat16),
            pltpu.VMEM((N_HOP, CHUNK, HALF), jnp.bfloat16),
            pltpu.VMEM((CHUNK, HALF), jnp.bfloat16),
            pltpu.VMEM((N_HOP, CHUNK, HALF), jnp.bfloat16),
            pltpu.VMEM((CHUNK, HALF), jnp.bfloat16),
            pltpu.VMEM((N_HOP, CHUNK, HALF), jnp.bfloat16),
            pltpu.SemaphoreType.DMA((2 * N_HOP,)),
            pltpu.SemaphoreType.DMA((2 * N_HOP,)),
            pltpu.SemaphoreType.DMA((2 * N_HOP,)),
            pltpu.SemaphoreType.DMA((2 * N_HOP,)),
        ],
        compiler_params=pltpu.CompilerParams(
            vmem_limit_bytes=100 * 1024 * 1024,
        ),
    )(xb, wq, wk, wv, wo, cos, sin, pmat)
    return out.reshape(1, SQ, D)
